# Initial kernel scaffold; baseline (speedup 1.0000x reference)
import functools
import jax
import jax.numpy as jnp
from jax import lax
from jax.experimental import pallas as pl
from jax.experimental.pallas import tpu as pltpu
from jax.experimental.pallas import tpu_sc as plsc


def _emb_lookup(idx_flat, table, *, n_per_w, chunk, nc):
    N = idx_flat.shape[0]
    D = table.shape[1]
    n_chunks = n_per_w // chunk
    ksub = chunk // 128

    mesh = plsc.VectorSubcoreMesh(core_axis_name="c", subcore_axis_name="s")

    @functools.partial(
        pl.kernel,
        out_type=jax.ShapeDtypeStruct((N, D), jnp.float32),
        mesh=mesh,
        scratch_types=[
            pltpu.VMEM((2, chunk), jnp.int32),
            pltpu.VMEM((2, chunk, D), jnp.float32),
            pltpu.SemaphoreType.DMA((2,)),
            pltpu.SemaphoreType.DMA((2,)),
            pltpu.SemaphoreType.DMA((2,)),
        ],
    )
    def emb(idx_hbm, table_hbm, out_hbm, idx_v, rows_v, sem_i, sem_g, sem_o):
        wid = lax.axis_index("s") * nc + lax.axis_index("c")
        base = wid * n_per_w

        def idx_copy(i, b):
            return pltpu.make_async_copy(
                idx_hbm.at[pl.ds(base + i * chunk, chunk)],
                idx_v.at[b],
                sem_i.at[b],
            )

        def gathers(b):
            return [
                pltpu.make_async_copy(
                    table_hbm.at[idx_v.at[b, pl.ds(j * 128, 128)]],
                    rows_v.at[b, pl.ds(j * 128, 128)],
                    sem_g.at[b],
                )
                for j in range(ksub)
            ]

        def out_copy(i, b):
            return pltpu.make_async_copy(
                rows_v.at[b],
                out_hbm.at[pl.ds(base + i * chunk, chunk)],
                sem_o.at[b],
            )

        idx_copy(0, 0).start()
        idx_copy(1, 1).start()

        def body(g, carry):
            for b in range(2):
                i = 2 * g + b
                idx_copy(i, b).wait()

                @pl.when(g > 0)
                def _():
                    out_copy(0, b).wait()

                gs = gathers(b)
                for d in gs:
                    d.start()
                for d in gs:
                    d.wait()

                @pl.when(i + 2 < n_chunks)
                def _():
                    idx_copy(i + 2, b).start()

                out_copy(i, b).start()
            return carry

        lax.fori_loop(0, n_chunks // 2, body, 0)
        out_copy(0, 0).wait()
        out_copy(0, 1).wait()

    return emb(idx_flat, table)


def kernel(indices, table):
    B, S = indices.shape
    N = B * S
    idx_flat = indices.reshape(N).astype(jnp.int32)

    info = plsc.get_sparse_core_info()
    nc, ns = info.num_cores, info.num_subcores
    nw = nc * ns
    n_per_w = N // nw

    out = _emb_lookup(idx_flat, table, n_per_w=n_per_w, chunk=1280, nc=nc)
    return out.reshape(B, S, table.shape[1])


# speedup vs baseline: 1.4937x; 1.4937x over previous
import functools
import jax
import jax.numpy as jnp
from jax import lax
from jax.experimental import pallas as pl
from jax.experimental.pallas import tpu as pltpu
from jax.experimental.pallas import tpu_sc as plsc


def _emb_lookup(idx_flat, table, *, n_per_w, chunk, nc):
    N = idx_flat.shape[0]
    D = table.shape[1]
    n_chunks = n_per_w // chunk
    ksub = chunk // 128

    mesh = plsc.VectorSubcoreMesh(core_axis_name="c", subcore_axis_name="s")

    @functools.partial(
        pl.kernel,
        out_type=jax.ShapeDtypeStruct((N, D), jnp.float32),
        mesh=mesh,
        compiler_params=pltpu.CompilerParams(use_tc_tiling_on_sc=False),
        scratch_types=[
            pltpu.VMEM((2, chunk), jnp.int32),
            pltpu.VMEM((2, chunk, D), jnp.float32),
            pltpu.SemaphoreType.DMA((2,)),
            pltpu.SemaphoreType.DMA((2,)),
            pltpu.SemaphoreType.DMA((2,)),
        ],
    )
    def emb(idx_hbm, table_hbm, out_hbm, idx_v, rows_v, sem_i, sem_g, sem_o):
        wid = lax.axis_index("s") * nc + lax.axis_index("c")
        base = wid * n_per_w

        def idx_copy(i, b):
            return pltpu.make_async_copy(
                idx_hbm.at[pl.ds(base + i * chunk, chunk)],
                idx_v.at[b],
                sem_i.at[b],
            )

        def gathers(b):
            return [
                pltpu.make_async_copy(
                    table_hbm.at[idx_v.at[b, pl.ds(j * 128, 128)]],
                    rows_v.at[b, pl.ds(j * 128, 128)],
                    sem_g.at[b],
                )
                for j in range(ksub)
            ]

        def out_copy(i, b):
            return pltpu.make_async_copy(
                rows_v.at[b],
                out_hbm.at[pl.ds(base + i * chunk, chunk)],
                sem_o.at[b],
            )

        idx_copy(0, 0).start()
        idx_copy(1, 1).start()

        def body(g, carry):
            for b in range(2):
                i = 2 * g + b
                idx_copy(i, b).wait()

                @pl.when(g > 0)
                def _():
                    out_copy(0, b).wait()

                gs = gathers(b)
                for d in gs:
                    d.start()
                for d in gs:
                    d.wait()

                @pl.when(i + 2 < n_chunks)
                def _():
                    idx_copy(i + 2, b).start()

                out_copy(i, b).start()
            return carry

        lax.fori_loop(0, n_chunks // 2, body, 0)
        out_copy(0, 0).wait()
        out_copy(0, 1).wait()

    return emb(idx_flat, table)


def kernel(indices, table):
    B, S = indices.shape
    N = B * S
    idx_flat = indices.reshape(N).astype(jnp.int32)

    info = plsc.get_sparse_core_info()
    nc, ns = info.num_cores, info.num_subcores
    nw = nc * ns
    n_per_w = N // nw

    out = _emb_lookup(idx_flat, table, n_per_w=n_per_w, chunk=1280, nc=nc)
    return out.reshape(B, S, table.shape[1])


# speedup vs baseline: 1.4941x; 1.0003x over previous
import functools
import jax
import jax.numpy as jnp
from jax import lax
from jax.experimental import pallas as pl
from jax.experimental.pallas import tpu as pltpu
from jax.experimental.pallas import tpu_sc as plsc


def _emb_lookup(idx_flat, table, *, n_per_w, chunk, sub, nc):
    N = idx_flat.shape[0]
    D = table.shape[1]
    n_chunks = n_per_w // chunk
    ksub = chunk // sub

    mesh = plsc.VectorSubcoreMesh(core_axis_name="c", subcore_axis_name="s")

    @functools.partial(
        pl.kernel,
        out_type=jax.ShapeDtypeStruct((N, D), jnp.float32),
        mesh=mesh,
        compiler_params=pltpu.CompilerParams(use_tc_tiling_on_sc=False),
        scratch_types=[
            pltpu.VMEM((n_per_w,), jnp.int32),
            pltpu.VMEM((2, chunk, D), jnp.float32),
            pltpu.SemaphoreType.DMA,
            pltpu.SemaphoreType.DMA((2,)),
            pltpu.SemaphoreType.DMA((2,)),
        ],
    )
    def emb(idx_hbm, table_hbm, out_hbm, idx_v, rows_v, sem_i, sem_g, sem_o):
        wid = lax.axis_index("s") * nc + lax.axis_index("c")
        base = wid * n_per_w

        pltpu.make_async_copy(
            idx_hbm.at[pl.ds(base, n_per_w)], idx_v, sem_i
        ).start()

        def gathers(i, b):
            return [
                pltpu.make_async_copy(
                    table_hbm.at[idx_v.at[pl.ds(i * chunk + j * sub, sub)]],
                    rows_v.at[b, pl.ds(j * sub, sub)],
                    sem_g.at[b],
                )
                for j in range(ksub)
            ]

        def out_copy(i, b):
            return pltpu.make_async_copy(
                rows_v.at[b],
                out_hbm.at[pl.ds(base + i * chunk, chunk)],
                sem_o.at[b],
            )

        pltpu.make_async_copy(
            idx_hbm.at[pl.ds(base, n_per_w)], idx_v, sem_i
        ).wait()

        def body(g, carry):
            for b in range(2):
                i = 2 * g + b

                @pl.when(g > 0)
                def _():
                    out_copy(0, b).wait()

                gs = gathers(i, b)
                for d in gs:
                    d.start()
                for d in gs:
                    d.wait()
                out_copy(i, b).start()
            return carry

        lax.fori_loop(0, n_chunks // 2, body, 0)
        out_copy(0, 0).wait()
        out_copy(0, 1).wait()

    return emb(idx_flat, table)


def kernel(indices, table):
    B, S = indices.shape
    N = B * S
    idx_flat = indices.reshape(N).astype(jnp.int32)

    info = plsc.get_sparse_core_info()
    nc, ns = info.num_cores, info.num_subcores
    nw = nc * ns
    n_per_w = N // nw

    out = _emb_lookup(
        idx_flat, table, n_per_w=n_per_w, chunk=1280, sub=128, nc=nc
    )
    return out.reshape(B, S, table.shape[1])
